# baseline (speedup 1.0000x reference)
import functools

import jax
import jax.numpy as jnp
from jax import lax
from jax.experimental import pallas as pl
from jax.experimental.pallas import tpu as pltpu
from jax.experimental.pallas import tpu_sc as plsc

N = 10000
E = 320000
D = 128
HD = D // 2
DE = 16
G = 32
OUTD = 64

NC = 2
NS = 16
NW = NC * NS
CH = 200
EPW = E // NW
NCH = EPW // CH
NCH2 = 2 * NCH
RPT = 624

_sc_mesh = plsc.VectorSubcoreMesh(
    core_axis_name="c", subcore_axis_name="s", num_cores=NC, num_subcores=NS)


def _wid():
    c = lax.axis_index("c")
    s = lax.axis_index("s")
    return c, s, c * NS + s


def _writeout(acc, out, c, s):
    @pl.when(s < NS - 1)
    def _most():
        pltpu.sync_copy(acc.at[pl.ds(s * RPT, RPT)],
                        out.at[c, pl.ds(s * RPT, RPT)])

    @pl.when(s == NS - 1)
    def _last():
        pltpu.sync_copy(acc.at[pl.ds((NS - 1) * RPT, N - (NS - 1) * RPT)],
                        out.at[c, pl.ds((NS - 1) * RPT, N - (NS - 1) * RPT)])


def _stage_idx(src3, dst3, idx_s, idx_d, s):
    pltpu.sync_copy(src3.at[s], idx_s.at[pl.ds(0, NCH)])
    pltpu.sync_copy(src3.at[NS + s], idx_s.at[pl.ds(NCH, NCH)])
    pltpu.sync_copy(dst3.at[s], idx_d.at[pl.ds(0, NCH)])
    pltpu.sync_copy(dst3.at[NS + s], idx_d.at[pl.ds(NCH, NCH)])


SL = 5
CPS = NCH // SL
EPS = EPW // SL


@functools.partial(
    pl.kernel,
    out_type=(
        jax.ShapeDtypeStruct((NC, N), jnp.float32),
        jax.ShapeDtypeStruct((NC, N), jnp.float32),
        jax.ShapeDtypeStruct((NC, N, DE), jnp.float32),
    ),
    mesh=_sc_mesh,
    compiler_params=pltpu.CompilerParams(use_tc_tiling_on_sc=False),
    scratch_types=[
        pltpu.VMEM((NCH, CH), jnp.int32),
        pltpu.VMEM((NCH, CH), jnp.int32),
        pltpu.VMEM((CH,), jnp.float32),
        pltpu.VMEM((2, EPS, DE), jnp.float32),
        pltpu.VMEM((2 * EPS,), jnp.float32),
        pltpu.VMEM_SHARED((N,), jnp.float32),
        pltpu.VMEM_SHARED((N,), jnp.float32),
        pltpu.VMEM_SHARED((N, DE), jnp.float32),
        pltpu.SemaphoreType.DMA,
        pltpu.SemaphoreType.DMA,
        pltpu.SemaphoreType.DMA,
        pltpu.SemaphoreType.DMA,
        pltpu.SemaphoreType.DMA,
        pltpu.SemaphoreType.DMA,
        pltpu.SemaphoreType.DMA,
        pltpu.SemaphoreType.DMA,
    ],
)
def _sc_pre(src3, dst3, ea, z1, z16, o_in, o_out, o_t,
            idx_s, idx_d, ones_v, ea_slab, dummy, in_acc, out_acc, t_acc,
            l0, l1, t0, t1, i0, i1, u0, u1):
    c, s, w = _wid()
    lsem = (l0, l1)
    tsem = (t0, t1)
    isem = (i0, i1)
    usem = (u0, u1)
    pltpu.sync_copy(src3.at[w], idx_s)
    pltpu.sync_copy(dst3.at[w], idx_d)
    for k in range(CH // 16):
        ones_v[pl.ds(k * 16, 16)] = jnp.ones((16,), jnp.float32)
    if CH % 16:
        ones_v[pl.ds(CH - 16, 16)] = jnp.ones((16,), jnp.float32)

    def slab_load(S, b):
        base = w * EPW + S * EPS
        return pltpu.make_async_copy(ea.at[pl.ds(base, EPS)], ea_slab.at[b],
                                     lsem[b])

    slab_load(0, 0).start()

    @pl.when(s == 0)
    def _zero():
        pltpu.sync_copy(z1, in_acc)
        pltpu.sync_copy(z1, out_acc)
        pltpu.sync_copy(z16, t_acc)

    plsc.subcore_barrier()

    def drain_slab(S):
        b = S % 2
        pltpu.make_async_copy(ea.at[pl.ds(w * EPW, EPS)], ea_slab.at[b],
                              tsem[b]).wait()
        pltpu.make_async_copy(z1.at[pl.ds(0, CPS * CH)],
                              dummy.at[pl.ds(0, CPS * CH)], isem[b]).wait()
        pltpu.make_async_copy(z1.at[pl.ds(0, CPS * CH)],
                              dummy.at[pl.ds(0, CPS * CH)], usem[b]).wait()

    for S in range(SL):
        b = S % 2
        slab_load(S, b).wait()
        if S >= 1:
            drain_slab(S - 1)
        if S + 1 < SL:
            slab_load(S + 1, 1 - b).start()

        def issue(k, carry, S=S, b=b):
            jt = S * CPS + k
            pltpu.async_copy(ea_slab.at[b, pl.ds(k * CH, CH)],
                             t_acc.at[idx_s.at[jt]], tsem[b], add=True)
            pltpu.async_copy(ones_v, in_acc.at[idx_d.at[jt]], isem[b],
                             add=True)
            pltpu.async_copy(ones_v, out_acc.at[idx_s.at[jt]], usem[b],
                             add=True)
            return carry

        lax.fori_loop(0, CPS, issue, 0)

    drain_slab(SL - 1)
    plsc.subcore_barrier()

    @pl.when(s == 0)
    def _write_counts():
        pltpu.sync_copy(in_acc, o_in.at[c])
        pltpu.sync_copy(out_acc, o_out.at[c])

    _writeout(t_acc, o_t, c, s)


NB = 2


def _zacc_pass(zp, c, idx_s, idx_d, gbuf, z_acc, gsems, ssems):
    def gather(j, b):
        return pltpu.make_async_copy(zp.at[c].at[idx_s.at[j]], gbuf.at[b],
                                     gsems[b])

    for b in range(NB):
        gather(b, b).start()

    def run(j, b):
        jt = jnp.asarray(j, jnp.int32)
        gather(jt, b).wait()
        sd = pltpu.async_copy(gbuf.at[b], z_acc.at[idx_d.at[jt]], ssems[b],
                              add=True)
        return sd

    def body(jj, carry):
        sdescs = []
        for b in range(NB):
            sdescs.append(run(NB * jj + b, b))
        for b in range(NB):
            j = NB * jj + b
            sdescs[b].wait()

            @pl.when(j + NB < NCH2)
            def _next(j=j, b=b):
                gather(j + NB, b).start()

        return carry

    lax.fori_loop(0, NCH2 // NB, body, 0)
    for b in range(NCH2 % NB):
        run(NCH2 - NCH2 % NB + b, b).wait()


@functools.partial(
    pl.kernel,
    out_type=(
        jax.ShapeDtypeStruct((NC, N, HD), jnp.float32),
        jax.ShapeDtypeStruct((NC, N, DE), jnp.float32),
    ),
    mesh=_sc_mesh,
    compiler_params=pltpu.CompilerParams(use_tc_tiling_on_sc=False),
    scratch_types=[
        pltpu.VMEM((NCH2, CH), jnp.int32),
        pltpu.VMEM((NCH2, CH), jnp.int32),
        pltpu.VMEM((2, CH), jnp.float32),
        pltpu.VMEM((2, CH, HD), jnp.float32),
        pltpu.VMEM((2, CH, DE), jnp.float32),
        pltpu.VMEM((2, CH, DE), jnp.float32),
        pltpu.VMEM_SHARED((N, HD), jnp.float32),
        pltpu.VMEM_SHARED((N, DE), jnp.float32),
        pltpu.SemaphoreType.DMA,
        pltpu.SemaphoreType.DMA,
        pltpu.SemaphoreType.DMA,
        pltpu.SemaphoreType.DMA,
        pltpu.SemaphoreType.DMA,
        pltpu.SemaphoreType.DMA,
        pltpu.SemaphoreType.DMA,
        pltpu.SemaphoreType.DMA,
        pltpu.SemaphoreType.DMA,
        pltpu.SemaphoreType.DMA,
    ],
)
def _sc_layer1(src3, dst3, ea, zp, dinv, z64, z16, o_z, o_s,
               idx_s, idx_d, dv_buf, gbuf, ea_buf, pay, z_acc, s_acc,
               g0, g1, s0, s1, d0, d1, a0, a1, p0, p1):
    c, s, w = _wid()
    gsems, ssems = (g0, g1), (s0, s1)
    dsems, asems, psems = (d0, d1), (a0, a1), (p0, p1)
    _stage_idx(src3, dst3, idx_s, idx_d, s)
    roff = c * NCH

    @pl.when(s == 0)
    def _zero():
        pltpu.sync_copy(z64, z_acc)
        pltpu.sync_copy(z16, s_acc)

    def dv_load(j, b):
        return pltpu.make_async_copy(dinv.at[idx_s.at[roff + j]],
                                     dv_buf.at[b], dsems[b])

    def ea_load(j, b):
        base = w * EPW + j * CH
        return pltpu.make_async_copy(ea.at[pl.ds(base, CH)], ea_buf.at[b],
                                     asems[b])

    def pay_scatter(j, b):
        return pltpu.make_async_copy(pay.at[b], s_acc.at[idx_d.at[roff + j]],
                                     psems[b])

    for b in range(2):
        dv_load(b, b).start()
        ea_load(b, b).start()

    plsc.subcore_barrier()

    def sacc_chunk(j, b):
        jt = jnp.asarray(j, jnp.int32)
        dv_load(jt, b).wait()
        ea_load(jt, b).wait()

        @pl.when(jt >= 2)
        def _prev():
            pay_scatter(jt, b).wait()

        def scale(i, c2):
            dv16 = dv_buf[b, pl.ds(i * 16, 16)]
            for k in range(16):
                r = i * 16 + k
                pay[b, r, :] = ea_buf[b, r, :] * dv16[k]
            return c2

        lax.fori_loop(0, CH // 16, scale, 0)
        if CH % 16:
            dv16 = dv_buf[b, pl.ds(CH - 16, 16)]
            for k in range(16):
                r = CH - 16 + k
                pay[b, r, :] = ea_buf[b, r, :] * dv16[k]
        pltpu.async_copy(pay.at[b], s_acc.at[idx_d.at[roff + jt]], psems[b],
                         add=True)

        @pl.when(jt + 2 < NCH)
        def _next():
            dv_load(jt + 2, b).start()
            ea_load(jt + 2, b).start()

    def sbody(jj, carry):
        for b in range(2):
            sacc_chunk(2 * jj + b, b)
        return carry

    lax.fori_loop(0, NCH // 2, sbody, 0)
    pay_scatter(jnp.asarray(NCH - 2, jnp.int32), 0).wait()
    pay_scatter(jnp.asarray(NCH - 1, jnp.int32), 1).wait()

    _zacc_pass(zp, c, idx_s, idx_d, gbuf, z_acc, gsems, ssems)
    plsc.subcore_barrier()
    _writeout(z_acc, o_z, c, s)
    _writeout(s_acc, o_s, c, s)


@functools.partial(
    pl.kernel,
    out_type=jax.ShapeDtypeStruct((NC, N, HD), jnp.float32),
    mesh=_sc_mesh,
    compiler_params=pltpu.CompilerParams(use_tc_tiling_on_sc=False),
    scratch_types=[
        pltpu.VMEM((NCH2, CH), jnp.int32),
        pltpu.VMEM((NCH2, CH), jnp.int32),
        pltpu.VMEM((2, CH, HD), jnp.float32),
        pltpu.VMEM_SHARED((N, HD), jnp.float32),
        pltpu.SemaphoreType.DMA,
        pltpu.SemaphoreType.DMA,
        pltpu.SemaphoreType.DMA,
        pltpu.SemaphoreType.DMA,
    ],
)
def _sc_layer2(src3, dst3, zp, z64, o_z, idx_s, idx_d, gbuf, z_acc,
               g0, g1, s0, s1):
    c, s, w = _wid()
    _stage_idx(src3, dst3, idx_s, idx_d, s)

    @pl.when(s == 0)
    def _zero():
        pltpu.sync_copy(z64, z_acc)

    plsc.subcore_barrier()
    _zacc_pass(zp, c, idx_s, idx_d, gbuf, z_acc, (g0, g1), (s0, s1))
    plsc.subcore_barrier()
    _writeout(z_acc, o_z, c, s)


_BLK = 1000
_GRID = N // _BLK
_PREC = lax.Precision.HIGHEST


def _tc_b_body(x_ref, w1n_ref, i0_ref, i1_ref, dinv_ref, zp1_ref):
    deg = i0_ref[...] + i1_ref[...] + 1.0
    dinv = lax.rsqrt(deg)
    dinv_ref[...] = dinv
    z = jnp.dot(x_ref[...], w1n_ref[...], preferred_element_type=jnp.float32,
                precision=_PREC)
    zp = dinv * z
    zp1_ref[0] = zp[:, :HD]
    zp1_ref[1] = zp[:, HD:]


def _tc_b(x, w1n, i0, i1):
    return pl.pallas_call(
        _tc_b_body,
        grid=(_GRID,),
        in_specs=[
            pl.BlockSpec((_BLK, D), lambda i: (i, 0)),
            pl.BlockSpec((D, D), lambda i: (0, 0)),
            pl.BlockSpec((_BLK, 1), lambda i: (i, 0)),
            pl.BlockSpec((_BLK, 1), lambda i: (i, 0)),
        ],
        out_specs=[
            pl.BlockSpec((_BLK, 1), lambda i: (i, 0)),
            pl.BlockSpec((NC, _BLK, HD), lambda i: (0, i, 0)),
        ],
        out_shape=[
            jax.ShapeDtypeStruct((N, 1), jnp.float32),
            jax.ShapeDtypeStruct((NC, N, HD), jnp.float32),
        ],
    )(x, w1n, i0, i1)


def _tc_d_body(zacc_ref, sacc_ref, zp1_ref, dinv_ref, w1e_ref, w2n_ref,
               zp2_ref):
    dinv = dinv_ref[...]
    ssum = sacc_ref[0] + sacc_ref[1]
    zacc = jnp.concatenate([zacc_ref[0], zacc_ref[1]], axis=1)
    zp1 = jnp.concatenate([zp1_ref[0], zp1_ref[1]], axis=1)
    pre = (zacc
           + jnp.dot(ssum, w1e_ref[...], preferred_element_type=jnp.float32,
                     precision=_PREC)
           + zp1)
    h1 = jnp.maximum(dinv * pre, 0.0)
    zp2 = dinv * jnp.dot(h1, w2n_ref[...], preferred_element_type=jnp.float32,
                         precision=_PREC)
    zp2_ref[0] = zp2[:, :HD]
    zp2_ref[1] = zp2[:, HD:]


def _tc_d(zacc, sacc, zp1, dinv, w1e, w2n):
    return pl.pallas_call(
        _tc_d_body,
        grid=(_GRID,),
        in_specs=[
            pl.BlockSpec((NC, _BLK, HD), lambda i: (0, i, 0)),
            pl.BlockSpec((NC, _BLK, DE), lambda i: (0, i, 0)),
            pl.BlockSpec((NC, _BLK, HD), lambda i: (0, i, 0)),
            pl.BlockSpec((_BLK, 1), lambda i: (i, 0)),
            pl.BlockSpec((DE, D), lambda i: (0, 0)),
            pl.BlockSpec((D, D), lambda i: (0, 0)),
        ],
        out_specs=pl.BlockSpec((NC, _BLK, HD), lambda i: (0, i, 0)),
        out_shape=jax.ShapeDtypeStruct((NC, N, HD), jnp.float32),
    )(zacc, sacc, zp1, dinv, w1e, w2n)


_PF = D + DE + 2


def _tc_f_body(zacc_ref, sacc_ref, zp2_ref, dinv_ref, t_ref, od_ref, b_ref,
               w2e_ref, we_ref, wc_ref, bc_ref, be_ref, out_ref, acc_ref):
    i = pl.program_id(0)

    @pl.when(i == 0)
    def _init():
        acc_ref[...] = jnp.zeros_like(acc_ref)

    dinv = dinv_ref[...]
    ssum = sacc_ref[0] + sacc_ref[1]
    zacc = jnp.concatenate([zacc_ref[0], zacc_ref[1]], axis=1)
    zp2 = jnp.concatenate([zp2_ref[0], zp2_ref[1]], axis=1)
    pre = (zacc
           + jnp.dot(ssum, w2e_ref[...], preferred_element_type=jnp.float32,
                     precision=_PREC)
           + zp2)
    h2 = jnp.maximum(dinv * pre, 0.0)
    tsum = t_ref[0] + t_ref[1]
    od = od_ref[0] + od_ref[1]
    ones = jnp.ones((_BLK, 1), jnp.float32)
    feats = jnp.concatenate([h2, tsum, od, ones], axis=1)
    onehot_t = (b_ref[...] ==
                lax.broadcasted_iota(jnp.int32, (_BLK, G), 1)
                ).astype(jnp.float32)
    acc_ref[...] += lax.dot_general(
        onehot_t, feats, (((0,), (0,)), ((), ())),
        preferred_element_type=jnp.float32, precision=_PREC)

    @pl.when(i == _GRID - 1)
    def _final():
        acc = acc_ref[...]
        ncnt = jnp.maximum(acc[:, _PF - 1:_PF], 1.0)
        ecnt = jnp.maximum(acc[:, _PF - 2:_PF - 1], 1.0)
        x_pool = acc[:, :D] / ncnt
        e_pool = (jnp.dot(acc[:, D:D + DE], we_ref[...],
                          preferred_element_type=jnp.float32,
                          precision=_PREC) / ecnt
                  + be_ref[...])
        hc = jnp.concatenate([x_pool, e_pool], axis=1)
        out_ref[...] = (jnp.dot(hc, wc_ref[...],
                                preferred_element_type=jnp.float32,
                                precision=_PREC)
                        + bc_ref[...])


def _tc_f(zacc, sacc, zp2, dinv, t, od, b, w2e, we, wc, bc, be):
    return pl.pallas_call(
        _tc_f_body,
        grid=(_GRID,),
        in_specs=[
            pl.BlockSpec((NC, _BLK, HD), lambda i: (0, i, 0)),
            pl.BlockSpec((NC, _BLK, DE), lambda i: (0, i, 0)),
            pl.BlockSpec((NC, _BLK, HD), lambda i: (0, i, 0)),
            pl.BlockSpec((_BLK, 1), lambda i: (i, 0)),
            pl.BlockSpec((NC, _BLK, DE), lambda i: (0, i, 0)),
            pl.BlockSpec((NC, _BLK, 1), lambda i: (0, i, 0)),
            pl.BlockSpec((_BLK, 1), lambda i: (i, 0)),
            pl.BlockSpec((DE, D), lambda i: (0, 0)),
            pl.BlockSpec((DE, D), lambda i: (0, 0)),
            pl.BlockSpec((2 * D, OUTD), lambda i: (0, 0)),
            pl.BlockSpec((1, OUTD), lambda i: (0, 0)),
            pl.BlockSpec((1, D), lambda i: (0, 0)),
        ],
        out_specs=pl.BlockSpec((G, OUTD), lambda i: (0, 0)),
        out_shape=jax.ShapeDtypeStruct((G, OUTD), jnp.float32),
        scratch_shapes=[pltpu.VMEM((G, _PF), jnp.float32)],
    )(zacc, sacc, zp2, dinv, t, od, b, w2e, we, wc, bc, be)


def kernel(x, edge_index, edge_attr, batch_index, W1n, W1e, W2n, W2e, We, be,
           Wc, bc):
    src3 = edge_index[0].reshape(NW, NCH, CH)
    dst3 = edge_index[1].reshape(NW, NCH, CH)
    z1 = jnp.zeros((N,), jnp.float32)
    z16 = jnp.zeros((N, DE), jnp.float32)
    z64 = jnp.zeros((N, HD), jnp.float32)

    indeg, outdeg, t_part = _sc_pre(src3, dst3, edge_attr, z1, z16)

    i0 = indeg[0].reshape(N, 1)
    i1 = indeg[1].reshape(N, 1)
    dinv, zp1 = _tc_b(x, W1n, i0, i1)

    zacc1, sacc = _sc_layer1(src3, dst3, edge_attr, zp1,
                             dinv.reshape(N), z64, z16)

    zp2 = _tc_d(zacc1, sacc, zp1, dinv, W1e, W2n)

    zacc2 = _sc_layer2(src3, dst3, zp2, z64)

    od3 = outdeg.reshape(NC, N, 1)
    return _tc_f(zacc2, sacc, zp2, dinv, t_part, od3,
                 batch_index.reshape(N, 1), W2e, We, Wc,
                 bc.reshape(1, OUTD), be.reshape(1, D))


# speedup vs baseline: 1.0729x; 1.0729x over previous
import functools

import jax
import jax.numpy as jnp
from jax import lax
from jax.experimental import pallas as pl
from jax.experimental.pallas import tpu as pltpu
from jax.experimental.pallas import tpu_sc as plsc

N = 10000
E = 320000
D = 128
HD = D // 2
DE = 16
G = 32
OUTD = 64

NC = 2
NS = 16
NW = NC * NS
CH = 80
EPW = E // NW
NCH = EPW // CH
NCH2 = 2 * NCH
RPT = 624

_sc_mesh = plsc.VectorSubcoreMesh(
    core_axis_name="c", subcore_axis_name="s", num_cores=NC, num_subcores=NS)


def _wid():
    c = lax.axis_index("c")
    s = lax.axis_index("s")
    return c, s, c * NS + s


def _writeout(acc, out, c, s):
    @pl.when(s < NS - 1)
    def _most():
        pltpu.sync_copy(acc.at[pl.ds(s * RPT, RPT)],
                        out.at[c, pl.ds(s * RPT, RPT)])

    @pl.when(s == NS - 1)
    def _last():
        pltpu.sync_copy(acc.at[pl.ds((NS - 1) * RPT, N - (NS - 1) * RPT)],
                        out.at[c, pl.ds((NS - 1) * RPT, N - (NS - 1) * RPT)])


def _stage_idx(src3, dst3, idx_s, idx_d, s):
    pltpu.sync_copy(src3.at[s], idx_s.at[pl.ds(0, NCH)])
    pltpu.sync_copy(src3.at[NS + s], idx_s.at[pl.ds(NCH, NCH)])
    pltpu.sync_copy(dst3.at[s], idx_d.at[pl.ds(0, NCH)])
    pltpu.sync_copy(dst3.at[NS + s], idx_d.at[pl.ds(NCH, NCH)])


SL = 5
CPS = NCH // SL
EPS = EPW // SL


@functools.partial(
    pl.kernel,
    out_type=(
        jax.ShapeDtypeStruct((NC, N), jnp.float32),
        jax.ShapeDtypeStruct((NC, N), jnp.float32),
        jax.ShapeDtypeStruct((NC, N, DE), jnp.float32),
    ),
    mesh=_sc_mesh,
    compiler_params=pltpu.CompilerParams(use_tc_tiling_on_sc=False),
    scratch_types=[
        pltpu.VMEM((NCH, CH), jnp.int32),
        pltpu.VMEM((NCH, CH), jnp.int32),
        pltpu.VMEM((CH,), jnp.float32),
        pltpu.VMEM((2, EPS, DE), jnp.float32),
        pltpu.VMEM((CPS * CH,), jnp.float32),
        pltpu.VMEM_SHARED((N,), jnp.float32),
        pltpu.VMEM_SHARED((N,), jnp.float32),
        pltpu.VMEM_SHARED((N, DE), jnp.float32),
        pltpu.SemaphoreType.DMA,
        pltpu.SemaphoreType.DMA,
        pltpu.SemaphoreType.DMA,
        pltpu.SemaphoreType.DMA,
        pltpu.SemaphoreType.DMA,
        pltpu.SemaphoreType.DMA,
        pltpu.SemaphoreType.DMA,
        pltpu.SemaphoreType.DMA,
    ],
)
def _sc_pre(src3, dst3, ea, z1, z16, o_in, o_out, o_t,
            idx_s, idx_d, ones_v, ea_slab, dummy, in_acc, out_acc, t_acc,
            l0, l1, t0, t1, i0, i1, u0, u1):
    c, s, w = _wid()
    lsem = (l0, l1)
    tsem = (t0, t1)
    isem = (i0, i1)
    usem = (u0, u1)
    pltpu.sync_copy(src3.at[w], idx_s)
    pltpu.sync_copy(dst3.at[w], idx_d)
    for k in range(CH // 16):
        ones_v[pl.ds(k * 16, 16)] = jnp.ones((16,), jnp.float32)

    def slab_load(S, b):
        base = w * EPW + S * EPS
        return pltpu.make_async_copy(ea.at[pl.ds(base, EPS)], ea_slab.at[b],
                                     lsem[b])

    slab_load(0, 0).start()

    @pl.when(s == 0)
    def _zero():
        pltpu.sync_copy(z1, in_acc)
        pltpu.sync_copy(z1, out_acc)
        pltpu.sync_copy(z16, t_acc)

    plsc.subcore_barrier()

    def drain_slab(S):
        b = S % 2
        pltpu.make_async_copy(ea.at[pl.ds(w * EPW, EPS)], ea_slab.at[b],
                              tsem[b]).wait()
        pltpu.make_async_copy(z1.at[pl.ds(0, CPS * CH)], dummy,
                              isem[b]).wait()
        pltpu.make_async_copy(z1.at[pl.ds(0, CPS * CH)], dummy,
                              usem[b]).wait()

    for S in range(SL):
        b = S % 2
        slab_load(S, b).wait()
        if S >= 1:
            drain_slab(S - 1)
        if S + 1 < SL:
            slab_load(S + 1, 1 - b).start()

        def issue(k, carry, S=S, b=b):
            jt = S * CPS + k
            pltpu.async_copy(ea_slab.at[b, pl.ds(k * CH, CH)],
                             t_acc.at[idx_s.at[jt]], tsem[b], add=True)
            pltpu.async_copy(ones_v, in_acc.at[idx_d.at[jt]], isem[b],
                             add=True)
            pltpu.async_copy(ones_v, out_acc.at[idx_s.at[jt]], usem[b],
                             add=True)
            return carry

        lax.fori_loop(0, CPS, issue, 0)

    drain_slab(SL - 1)
    plsc.subcore_barrier()

    @pl.when(s == 0)
    def _write_counts():
        pltpu.sync_copy(in_acc, o_in.at[c])
        pltpu.sync_copy(out_acc, o_out.at[c])

    _writeout(t_acc, o_t, c, s)


NB = 4


def _zacc_pass(zp, c, idx_s, idx_d, gbuf, z_acc, gsems, ssems):
    def gather(j, b):
        return pltpu.make_async_copy(zp.at[c].at[idx_s.at[j]], gbuf.at[b],
                                     gsems[b])

    for b in range(NB):
        gather(b, b).start()

    def run(j, b):
        jt = jnp.asarray(j, jnp.int32)
        gather(jt, b).wait()
        sd = pltpu.async_copy(gbuf.at[b], z_acc.at[idx_d.at[jt]], ssems[b],
                              add=True)
        return sd

    def body(jj, carry):
        sdescs = []
        for b in range(NB):
            sdescs.append(run(NB * jj + b, b))
        for b in range(NB):
            j = NB * jj + b
            sdescs[b].wait()

            @pl.when(j + NB < NCH2)
            def _next(j=j, b=b):
                gather(j + NB, b).start()

        return carry

    lax.fori_loop(0, NCH2 // NB, body, 0)
    for b in range(NCH2 % NB):
        run(NCH2 - NCH2 % NB + b, b).wait()


@functools.partial(
    pl.kernel,
    out_type=(
        jax.ShapeDtypeStruct((NC, N, HD), jnp.float32),
        jax.ShapeDtypeStruct((NC, N, DE), jnp.float32),
    ),
    mesh=_sc_mesh,
    compiler_params=pltpu.CompilerParams(use_tc_tiling_on_sc=False),
    scratch_types=[
        pltpu.VMEM((NCH2, CH), jnp.int32),
        pltpu.VMEM((NCH2, CH), jnp.int32),
        pltpu.VMEM((2, CH), jnp.float32),
        pltpu.VMEM((4, CH, HD), jnp.float32),
        pltpu.VMEM((2, CH, DE), jnp.float32),
        pltpu.VMEM((2, CH, DE), jnp.float32),
        pltpu.VMEM_SHARED((N, HD), jnp.float32),
        pltpu.VMEM_SHARED((N, DE), jnp.float32),
        pltpu.SemaphoreType.DMA,
        pltpu.SemaphoreType.DMA,
        pltpu.SemaphoreType.DMA,
        pltpu.SemaphoreType.DMA,
        pltpu.SemaphoreType.DMA,
        pltpu.SemaphoreType.DMA,
        pltpu.SemaphoreType.DMA,
        pltpu.SemaphoreType.DMA,
        pltpu.SemaphoreType.DMA,
        pltpu.SemaphoreType.DMA,
        pltpu.SemaphoreType.DMA,
        pltpu.SemaphoreType.DMA,
        pltpu.SemaphoreType.DMA,
        pltpu.SemaphoreType.DMA,
    ],
)
def _sc_layer1(src3, dst3, ea, zp, dinv, z64, z16, o_z, o_s,
               idx_s, idx_d, dv_buf, gbuf, ea_buf, pay, z_acc, s_acc,
               g0, g1, g2, g3, s0, s1, s2, s3, d0, d1, a0, a1, p0, p1):
    c, s, w = _wid()
    gsems, ssems = (g0, g1, g2, g3), (s0, s1, s2, s3)
    dsems, asems, psems = (d0, d1), (a0, a1), (p0, p1)
    _stage_idx(src3, dst3, idx_s, idx_d, s)
    roff = c * NCH

    @pl.when(s == 0)
    def _zero():
        pltpu.sync_copy(z64, z_acc)
        pltpu.sync_copy(z16, s_acc)

    def dv_load(j, b):
        return pltpu.make_async_copy(dinv.at[idx_s.at[roff + j]],
                                     dv_buf.at[b], dsems[b])

    def ea_load(j, b):
        base = w * EPW + j * CH
        return pltpu.make_async_copy(ea.at[pl.ds(base, CH)], ea_buf.at[b],
                                     asems[b])

    def pay_scatter(j, b):
        return pltpu.make_async_copy(pay.at[b], s_acc.at[idx_d.at[roff + j]],
                                     psems[b])

    for b in range(2):
        dv_load(b, b).start()
        ea_load(b, b).start()

    plsc.subcore_barrier()

    def sacc_chunk(j, b):
        jt = jnp.asarray(j, jnp.int32)
        dv_load(jt, b).wait()
        ea_load(jt, b).wait()

        @pl.when(jt >= 2)
        def _prev():
            pay_scatter(jt, b).wait()

        def scale(i, c2):
            dv16 = dv_buf[b, pl.ds(i * 16, 16)]
            for k in range(16):
                r = i * 16 + k
                pay[b, r, :] = ea_buf[b, r, :] * dv16[k]
            return c2

        lax.fori_loop(0, CH // 16, scale, 0)
        pltpu.async_copy(pay.at[b], s_acc.at[idx_d.at[roff + jt]], psems[b],
                         add=True)

        @pl.when(jt + 2 < NCH)
        def _next():
            dv_load(jt + 2, b).start()
            ea_load(jt + 2, b).start()

    def sbody(jj, carry):
        for b in range(2):
            sacc_chunk(2 * jj + b, b)
        return carry

    lax.fori_loop(0, NCH // 2, sbody, 0)
    sacc_chunk(124, 0)
    pay_scatter(jnp.asarray(123, jnp.int32), 1).wait()
    pay_scatter(jnp.asarray(124, jnp.int32), 0).wait()

    _zacc_pass(zp, c, idx_s, idx_d, gbuf, z_acc, gsems, ssems)
    plsc.subcore_barrier()
    _writeout(z_acc, o_z, c, s)
    _writeout(s_acc, o_s, c, s)


@functools.partial(
    pl.kernel,
    out_type=jax.ShapeDtypeStruct((NC, N, HD), jnp.float32),
    mesh=_sc_mesh,
    compiler_params=pltpu.CompilerParams(use_tc_tiling_on_sc=False),
    scratch_types=[
        pltpu.VMEM((NCH2, CH), jnp.int32),
        pltpu.VMEM((NCH2, CH), jnp.int32),
        pltpu.VMEM((4, CH, HD), jnp.float32),
        pltpu.VMEM_SHARED((N, HD), jnp.float32),
        pltpu.SemaphoreType.DMA,
        pltpu.SemaphoreType.DMA,
        pltpu.SemaphoreType.DMA,
        pltpu.SemaphoreType.DMA,
        pltpu.SemaphoreType.DMA,
        pltpu.SemaphoreType.DMA,
        pltpu.SemaphoreType.DMA,
        pltpu.SemaphoreType.DMA,
    ],
)
def _sc_layer2(src3, dst3, zp, z64, o_z, idx_s, idx_d, gbuf, z_acc,
               g0, g1, g2, g3, s0, s1, s2, s3):
    c, s, w = _wid()
    _stage_idx(src3, dst3, idx_s, idx_d, s)

    @pl.when(s == 0)
    def _zero():
        pltpu.sync_copy(z64, z_acc)

    plsc.subcore_barrier()
    _zacc_pass(zp, c, idx_s, idx_d, gbuf, z_acc, (g0, g1, g2, g3),
               (s0, s1, s2, s3))
    plsc.subcore_barrier()
    _writeout(z_acc, o_z, c, s)


_BLK = 1000
_GRID = N // _BLK
_PREC = lax.Precision.HIGHEST


def _tc_b_body(x_ref, w1n_ref, i0_ref, i1_ref, dinv_ref, zp1_ref):
    deg = i0_ref[...] + i1_ref[...] + 1.0
    dinv = lax.rsqrt(deg)
    dinv_ref[...] = dinv
    z = jnp.dot(x_ref[...], w1n_ref[...], preferred_element_type=jnp.float32,
                precision=_PREC)
    zp = dinv * z
    zp1_ref[0] = zp[:, :HD]
    zp1_ref[1] = zp[:, HD:]


def _tc_b(x, w1n, i0, i1):
    return pl.pallas_call(
        _tc_b_body,
        grid=(_GRID,),
        in_specs=[
            pl.BlockSpec((_BLK, D), lambda i: (i, 0)),
            pl.BlockSpec((D, D), lambda i: (0, 0)),
            pl.BlockSpec((_BLK, 1), lambda i: (i, 0)),
            pl.BlockSpec((_BLK, 1), lambda i: (i, 0)),
        ],
        out_specs=[
            pl.BlockSpec((_BLK, 1), lambda i: (i, 0)),
            pl.BlockSpec((NC, _BLK, HD), lambda i: (0, i, 0)),
        ],
        out_shape=[
            jax.ShapeDtypeStruct((N, 1), jnp.float32),
            jax.ShapeDtypeStruct((NC, N, HD), jnp.float32),
        ],
    )(x, w1n, i0, i1)


def _tc_d_body(zacc_ref, sacc_ref, zp1_ref, dinv_ref, w1e_ref, w2n_ref,
               zp2_ref):
    dinv = dinv_ref[...]
    ssum = sacc_ref[0] + sacc_ref[1]
    zacc = jnp.concatenate([zacc_ref[0], zacc_ref[1]], axis=1)
    zp1 = jnp.concatenate([zp1_ref[0], zp1_ref[1]], axis=1)
    pre = (zacc
           + jnp.dot(ssum, w1e_ref[...], preferred_element_type=jnp.float32,
                     precision=_PREC)
           + zp1)
    h1 = jnp.maximum(dinv * pre, 0.0)
    zp2 = dinv * jnp.dot(h1, w2n_ref[...], preferred_element_type=jnp.float32,
                         precision=_PREC)
    zp2_ref[0] = zp2[:, :HD]
    zp2_ref[1] = zp2[:, HD:]


def _tc_d(zacc, sacc, zp1, dinv, w1e, w2n):
    return pl.pallas_call(
        _tc_d_body,
        grid=(_GRID,),
        in_specs=[
            pl.BlockSpec((NC, _BLK, HD), lambda i: (0, i, 0)),
            pl.BlockSpec((NC, _BLK, DE), lambda i: (0, i, 0)),
            pl.BlockSpec((NC, _BLK, HD), lambda i: (0, i, 0)),
            pl.BlockSpec((_BLK, 1), lambda i: (i, 0)),
            pl.BlockSpec((DE, D), lambda i: (0, 0)),
            pl.BlockSpec((D, D), lambda i: (0, 0)),
        ],
        out_specs=pl.BlockSpec((NC, _BLK, HD), lambda i: (0, i, 0)),
        out_shape=jax.ShapeDtypeStruct((NC, N, HD), jnp.float32),
    )(zacc, sacc, zp1, dinv, w1e, w2n)


_PF = D + DE + 2


def _tc_f_body(zacc_ref, sacc_ref, zp2_ref, dinv_ref, t_ref, od_ref, b_ref,
               w2e_ref, we_ref, wc_ref, bc_ref, be_ref, out_ref, acc_ref):
    i = pl.program_id(0)

    @pl.when(i == 0)
    def _init():
        acc_ref[...] = jnp.zeros_like(acc_ref)

    dinv = dinv_ref[...]
    ssum = sacc_ref[0] + sacc_ref[1]
    zacc = jnp.concatenate([zacc_ref[0], zacc_ref[1]], axis=1)
    zp2 = jnp.concatenate([zp2_ref[0], zp2_ref[1]], axis=1)
    pre = (zacc
           + jnp.dot(ssum, w2e_ref[...], preferred_element_type=jnp.float32,
                     precision=_PREC)
           + zp2)
    h2 = jnp.maximum(dinv * pre, 0.0)
    tsum = t_ref[0] + t_ref[1]
    od = od_ref[0] + od_ref[1]
    ones = jnp.ones((_BLK, 1), jnp.float32)
    feats = jnp.concatenate([h2, tsum, od, ones], axis=1)
    onehot_t = (b_ref[...] ==
                lax.broadcasted_iota(jnp.int32, (_BLK, G), 1)
                ).astype(jnp.float32)
    acc_ref[...] += lax.dot_general(
        onehot_t, feats, (((0,), (0,)), ((), ())),
        preferred_element_type=jnp.float32, precision=_PREC)

    @pl.when(i == _GRID - 1)
    def _final():
        acc = acc_ref[...]
        ncnt = jnp.maximum(acc[:, _PF - 1:_PF], 1.0)
        ecnt = jnp.maximum(acc[:, _PF - 2:_PF - 1], 1.0)
        x_pool = acc[:, :D] / ncnt
        e_pool = (jnp.dot(acc[:, D:D + DE], we_ref[...],
                          preferred_element_type=jnp.float32,
                          precision=_PREC) / ecnt
                  + be_ref[...])
        hc = jnp.concatenate([x_pool, e_pool], axis=1)
        out_ref[...] = (jnp.dot(hc, wc_ref[...],
                                preferred_element_type=jnp.float32,
                                precision=_PREC)
                        + bc_ref[...])


def _tc_f(zacc, sacc, zp2, dinv, t, od, b, w2e, we, wc, bc, be):
    return pl.pallas_call(
        _tc_f_body,
        grid=(_GRID,),
        in_specs=[
            pl.BlockSpec((NC, _BLK, HD), lambda i: (0, i, 0)),
            pl.BlockSpec((NC, _BLK, DE), lambda i: (0, i, 0)),
            pl.BlockSpec((NC, _BLK, HD), lambda i: (0, i, 0)),
            pl.BlockSpec((_BLK, 1), lambda i: (i, 0)),
            pl.BlockSpec((NC, _BLK, DE), lambda i: (0, i, 0)),
            pl.BlockSpec((NC, _BLK, 1), lambda i: (0, i, 0)),
            pl.BlockSpec((_BLK, 1), lambda i: (i, 0)),
            pl.BlockSpec((DE, D), lambda i: (0, 0)),
            pl.BlockSpec((DE, D), lambda i: (0, 0)),
            pl.BlockSpec((2 * D, OUTD), lambda i: (0, 0)),
            pl.BlockSpec((1, OUTD), lambda i: (0, 0)),
            pl.BlockSpec((1, D), lambda i: (0, 0)),
        ],
        out_specs=pl.BlockSpec((G, OUTD), lambda i: (0, 0)),
        out_shape=jax.ShapeDtypeStruct((G, OUTD), jnp.float32),
        scratch_shapes=[pltpu.VMEM((G, _PF), jnp.float32)],
    )(zacc, sacc, zp2, dinv, t, od, b, w2e, we, wc, bc, be)


def kernel(x, edge_index, edge_attr, batch_index, W1n, W1e, W2n, W2e, We, be,
           Wc, bc):
    src3 = edge_index[0].reshape(NW, NCH, CH)
    dst3 = edge_index[1].reshape(NW, NCH, CH)
    z1 = jnp.zeros((N,), jnp.float32)
    z16 = jnp.zeros((N, DE), jnp.float32)
    z64 = jnp.zeros((N, HD), jnp.float32)

    indeg, outdeg, t_part = _sc_pre(src3, dst3, edge_attr, z1, z16)

    i0 = indeg[0].reshape(N, 1)
    i1 = indeg[1].reshape(N, 1)
    dinv, zp1 = _tc_b(x, W1n, i0, i1)

    zacc1, sacc = _sc_layer1(src3, dst3, edge_attr, zp1,
                             dinv.reshape(N), z64, z16)

    zp2 = _tc_d(zacc1, sacc, zp1, dinv, W1e, W2n)

    zacc2 = _sc_layer2(src3, dst3, zp2, z64)

    od3 = outdeg.reshape(NC, N, 1)
    return _tc_f(zacc2, sacc, zp2, dinv, t_part, od3,
                 batch_index.reshape(N, 1), W2e, We, Wc,
                 bc.reshape(1, OUTD), be.reshape(1, D))
